# baseline (speedup 1.0000x reference)
import jax
import jax.numpy as jnp
from jax import lax
from jax.experimental import pallas as pl
from jax.experimental.pallas import tpu as pltpu
from jax.experimental.pallas import tpu_sc as plsc
import functools

_N = 10
_D = 256


def _sc_body(q_hbm, idx_hbm, out_hbm, idx_v, rows_v, self_v, out_v, sem):
    cid = lax.axis_index("c")
    sid = lax.axis_index("s")

    @pl.when(jnp.logical_and(cid == 0, sid == 0))
    def _():
        pltpu.sync_copy(idx_hbm, idx_v)
        pltpu.async_copy(q_hbm.at[idx_v], rows_v, sem).wait()
        pltpu.sync_copy(q_hbm, self_v)
        for i in range(_N):
            for c in range(_D // 16):
                sl = pl.ds(16 * c, 16)
                acc = (self_v[i, sl] + rows_v[3 * i, sl]
                       + rows_v[3 * i + 1, sl] + rows_v[3 * i + 2, sl])
                out_v[i, sl] = acc * 0.25
        pltpu.sync_copy(out_v, out_hbm)


_sc_gather_mean = functools.partial(
    pl.kernel,
    out_type=jax.ShapeDtypeStruct((_N, _D), jnp.float32),
    mesh=plsc.VectorSubcoreMesh(core_axis_name="c", subcore_axis_name="s"),
    scratch_types=[
        pltpu.VMEM((30,), jnp.int32),
        pltpu.VMEM((30, _D), jnp.float32),
        pltpu.VMEM((_N, _D), jnp.float32),
        pltpu.VMEM((_N, _D), jnp.float32),
        pltpu.SemaphoreType.DMA,
    ],
)(_sc_body)


def _tc_body(sp_v, st_v, wc_v, wa_v, bc_v, ba_v, out1_ref, q_ref):
    sp = sp_v[...]
    st = st_v[...]

    out1 = jax.lax.dot_general(sp, wc_v[0:64, :],
                               (((1,), (0,)), ((), ())),
                               preferred_element_type=jnp.float32)
    out1 += jax.lax.dot_general(st, wc_v[64:195, :],
                                (((1,), (0,)), ((), ())),
                                preferred_element_type=jnp.float32)
    out1_ref[...] = out1 + bc_v[...]

    q = jax.lax.dot_general(st, wa_v[...],
                            (((1,), (0,)), ((), ())),
                            preferred_element_type=jnp.float32)
    q_ref[...] = q + ba_v[...]


@jax.jit
def kernel(spatial, structural, neighbour, W_comb, b_comb, W_agg, b_agg):
    idx = neighbour.astype(jnp.int32).reshape(30)

    out_shape = (jax.ShapeDtypeStruct((_N, 256), jnp.float32),
                 jax.ShapeDtypeStruct((_N, 256), jnp.float32))
    vmem_spec = pl.BlockSpec(memory_space=pltpu.VMEM)
    out1, q = pl.pallas_call(
        _tc_body,
        out_shape=out_shape,
        in_specs=[vmem_spec] * 6,
    )(spatial, structural, W_comb.T, W_agg.T,
      b_comb.reshape(1, 256), b_agg.reshape(1, 256))

    out2 = _sc_gather_mean(q, idx)
    return (out1, out2)
